# baseline (speedup 1.0000x reference)
import functools

import jax
import jax.numpy as jnp
from jax import lax
from jax.experimental import pallas as pl
from jax.experimental.pallas import tpu as pltpu
from jax.experimental.pallas import tpu_sc as plsc

B = 16384
D = 32
NC, NS = 2, 16
NW = NC * NS
BPW = B // NW
CHUNK = 128
CPW = BPW // CHUNK


def _sc_gather_body(uidx_hbm, iidx_hbm, gu_hbm, gi_hbm, mu_hbm, mi_hbm,
                    gu_out, gi_out, mu_out, mi_out,
                    uidx_v, iidx_v, gu_v, gi_v, mu_v, mi_v, sem):
    wid = lax.axis_index("s") * NC + lax.axis_index("c")
    cbase = wid * CPW
    pltpu.sync_copy(uidx_hbm.at[pl.ds(cbase, CPW)], uidx_v)
    pltpu.sync_copy(iidx_hbm.at[pl.ds(cbase, CPW)], iidx_v)
    copies = []
    for tab, idxv, buf in ((gu_hbm, uidx_v, gu_v), (gi_hbm, iidx_v, gi_v),
                           (mu_hbm, uidx_v, mu_v), (mi_hbm, iidx_v, mi_v)):
        for c in range(CPW):
            copies.append(pltpu.async_copy(
                tab.at[idxv.at[c]], buf.at[pl.ds(c * CHUNK, CHUNK)], sem))
    for cp in copies:
        cp.wait()
    rbase = wid * BPW
    for buf, out in ((gu_v, gu_out), (gi_v, gi_out),
                     (mu_v, mu_out), (mi_v, mi_out)):
        pltpu.sync_copy(buf, out.at[pl.ds(rbase, BPW)])


_ROWS = jax.ShapeDtypeStruct((B, D), jnp.float32)

_sc_gather = functools.partial(
    pl.kernel,
    out_type=(_ROWS, _ROWS, _ROWS, _ROWS),
    mesh=plsc.VectorSubcoreMesh(
        core_axis_name="c", subcore_axis_name="s",
        num_cores=NC, num_subcores=NS),
    scratch_types=(
        pltpu.VMEM((CPW, CHUNK), jnp.int32),
        pltpu.VMEM((CPW, CHUNK), jnp.int32),
        pltpu.VMEM((BPW, D), jnp.float32),
        pltpu.VMEM((BPW, D), jnp.float32),
        pltpu.VMEM((BPW, D), jnp.float32),
        pltpu.VMEM((BPW, D), jnp.float32),
        pltpu.SemaphoreType.DMA,
    ),
    compiler_params=pltpu.CompilerParams(use_tc_tiling_on_sc=False),
)(_sc_gather_body)


def _tc_body(gu_ref, gi_ref, mu_ref, mi_ref, w1_ref, w2_ref, b1_ref,
             pg_ref, pm_ref, pb_ref, out_ref):
    h = jnp.dot(mu_ref[...], w1_ref[...], preferred_element_type=jnp.float32)
    h = h + jnp.dot(mi_ref[...], w2_ref[...],
                    preferred_element_type=jnp.float32)
    h = jnp.maximum(h + b1_ref[...], 0.0)
    gmf = gu_ref[...] * gi_ref[...]
    out_ref[...] = (jnp.sum(gmf * pg_ref[...], axis=1, keepdims=True)
                    + jnp.sum(h * pm_ref[...], axis=1, keepdims=True)
                    + pb_ref[...])


_BLK = 2048


def kernel(user_indices, item_indices, gmf_user_emb, gmf_item_emb,
           mlp_user_emb, mlp_item_emb, fc1_w, fc1_b, pred_w, pred_b):
    uidx = user_indices.astype(jnp.int32).reshape(B // CHUNK, CHUNK)
    iidx = item_indices.astype(jnp.int32).reshape(B // CHUNK, CHUNK)
    gu, gi, mu, mi = _sc_gather(uidx, iidx, gmf_user_emb, gmf_item_emb,
                                mlp_user_emb, mlp_item_emb)

    w1 = fc1_w[:, :D].T
    w2 = fc1_w[:, D:].T
    b1 = fc1_b.reshape(1, D)
    pg = pred_w[:, :D]
    pm = pred_w[:, D:]
    pb = pred_b.reshape(1, 1)

    rows_spec = pl.BlockSpec((_BLK, D), lambda i: (i, 0))
    w_spec = pl.BlockSpec((D, D), lambda i: (0, 0))
    v_spec = pl.BlockSpec((1, D), lambda i: (0, 0))
    s_spec = pl.BlockSpec((1, 1), lambda i: (0, 0))
    out2d = pl.pallas_call(
        _tc_body,
        grid=(B // _BLK,),
        in_specs=[rows_spec, rows_spec, rows_spec, rows_spec,
                  w_spec, w_spec, v_spec, v_spec, v_spec, s_spec],
        out_specs=pl.BlockSpec((_BLK, 1), lambda i: (i, 0)),
        out_shape=jax.ShapeDtypeStruct((B, 1), jnp.float32),
    )(gu, gi, mu, mi, w1, w2, b1, pg, pm, pb)
    return jnp.squeeze(out2d, axis=-1)


# speedup vs baseline: 3.1336x; 3.1336x over previous
import functools

import jax
import jax.numpy as jnp
from jax import lax
from jax.experimental import pallas as pl
from jax.experimental.pallas import tpu as pltpu
from jax.experimental.pallas import tpu_sc as plsc

B = 16384
D = 32
V = 1000000
NC, NS = 2, 16
NW = NC * NS
BPW = B // NW
LANE = 128
LAST = (V // LANE) * LANE
TAIL = V - LAST


def _sc_gather_body(ut_g, it_g, ut_m, it_m, aug, aig, aum, aim,
                    uidx_hbm, iidx_hbm,
                    gu_out, gi_out, mu_out, mi_out,
                    uidx_v, iidx_v,
                    bug0, bug1, big0, big1, bum0, bum1, bim0, bim1,
                    ou, oi, om, on, sem0, sem1):
    wid = lax.axis_index("s") * NC + lax.axis_index("c")
    base = wid * BPW
    pltpu.sync_copy(uidx_hbm.at[pl.ds(base, BPW)], uidx_v.at[pl.ds(0, BPW)])
    pltpu.sync_copy(iidx_hbm.at[pl.ds(base, BPW)], iidx_v.at[pl.ds(0, BPW)])

    def fetch(tab, aux, off, blk, sem, start):
        @pl.when(off < LAST)
        def _full():
            o = pl.multiple_of(off, LANE)
            cp = pltpu.make_async_copy(tab.at[:, pl.ds(o, LANE)], blk, sem)
            cp.start() if start else cp.wait()

        @pl.when(off >= LAST)
        def _tail():
            cp = pltpu.make_async_copy(aux, blk, sem)
            cp.start() if start else cp.wait()

    def quad(u, i, blks, sem, start):
        uo = u & ~(LANE - 1)
        io = i & ~(LANE - 1)
        fetch(ut_g, aug, uo, blks[0], sem, start)
        fetch(it_g, aig, io, blks[1], sem, start)
        fetch(ut_m, aum, uo, blks[2], sem, start)
        fetch(it_m, aim, io, blks[3], sem, start)

    def extract(u, i, j, blks):
        rows = lax.iota(jnp.int32, 16)
        ul = jnp.full((16,), u & (LANE - 1), jnp.int32)
        il = jnp.full((16,), i & (LANE - 1), jnp.int32)
        col = jnp.full((16,), j, jnp.int32)
        for blk, lane, out in ((blks[0], ul, ou), (blks[1], il, oi),
                               (blks[2], ul, om), (blks[3], il, on)):
            for half in (0, 16):
                r = rows + half
                v = plsc.load_gather(blk, [r, lane])
                plsc.store_scatter(out, [r, col], v)

    ph0 = (bug0, big0, bum0, bim0)
    ph1 = (bug1, big1, bum1, bim1)
    uv0 = uidx_v[pl.ds(0, 16)]
    iv0 = iidx_v[pl.ds(0, 16)]
    quad(uv0[0], iv0[0], ph0, sem0, True)
    quad(uv0[1], iv0[1], ph1, sem1, True)

    def step(j2, _):
        j0 = j2 * 2
        uv = uidx_v[pl.ds(j0, 16)]
        iv = iidx_v[pl.ds(j0, 16)]

        quad(uv[0], iv[0], ph0, sem0, False)
        extract(uv[0], iv[0], j0, ph0)

        @pl.when(j0 + 2 < BPW)
        def _i0():
            quad(uv[2], iv[2], ph0, sem0, True)

        quad(uv[1], iv[1], ph1, sem1, False)
        extract(uv[1], iv[1], j0 + 1, ph1)

        @pl.when(j0 + 3 < BPW)
        def _i1():
            quad(uv[3], iv[3], ph1, sem1, True)
        return 0

    lax.fori_loop(0, BPW // 2, step, 0)

    for buf, out in ((ou, gu_out), (oi, gi_out), (om, mu_out), (on, mi_out)):
        pltpu.sync_copy(buf, out.at[:, pl.ds(base, BPW)])


_FM = jax.ShapeDtypeStruct((D, B), jnp.float32)
_BLKT = pltpu.VMEM((D, LANE), jnp.float32)

_sc_gather = functools.partial(
    pl.kernel,
    out_type=(_FM, _FM, _FM, _FM),
    mesh=plsc.VectorSubcoreMesh(
        core_axis_name="c", subcore_axis_name="s",
        num_cores=NC, num_subcores=NS),
    scratch_types=(
        pltpu.VMEM((BPW + 16,), jnp.int32),
        pltpu.VMEM((BPW + 16,), jnp.int32),
        _BLKT, _BLKT, _BLKT, _BLKT, _BLKT, _BLKT, _BLKT, _BLKT,
        pltpu.VMEM((D, BPW), jnp.float32),
        pltpu.VMEM((D, BPW), jnp.float32),
        pltpu.VMEM((D, BPW), jnp.float32),
        pltpu.VMEM((D, BPW), jnp.float32),
        pltpu.SemaphoreType.DMA,
        pltpu.SemaphoreType.DMA,
    ),
    compiler_params=pltpu.CompilerParams(needs_layout_passes=False),
)(_sc_gather_body)


def _tc_body(gu_ref, gi_ref, mu_ref, mi_ref, w1_ref, w2_ref, b1_ref,
             pg_ref, pm_ref, pb_ref, out_ref):
    h = jnp.dot(w1_ref[...], mu_ref[...], preferred_element_type=jnp.float32)
    h = h + jnp.dot(w2_ref[...], mi_ref[...],
                    preferred_element_type=jnp.float32)
    h = jnp.maximum(h + b1_ref[...], 0.0)
    gmf = gu_ref[...] * gi_ref[...]
    out_ref[...] = (jnp.dot(pg_ref[...], gmf,
                            preferred_element_type=jnp.float32)
                    + jnp.dot(pm_ref[...], h,
                              preferred_element_type=jnp.float32)
                    + pb_ref[...])


_BLK = 2048


def kernel(user_indices, item_indices, gmf_user_emb, gmf_item_emb,
           mlp_user_emb, mlp_item_emb, fc1_w, fc1_b, pred_w, pred_b):
    uidx = user_indices.astype(jnp.int32)
    iidx = item_indices.astype(jnp.int32)
    def _aux(tab):
        return jnp.pad(tab.T[:, LAST:], ((0, 0), (0, LANE - TAIL)))

    gu, gi, mu, mi = _sc_gather(
        gmf_user_emb.T, gmf_item_emb.T, mlp_user_emb.T, mlp_item_emb.T,
        _aux(gmf_user_emb), _aux(gmf_item_emb),
        _aux(mlp_user_emb), _aux(mlp_item_emb), uidx, iidx)

    w1 = fc1_w[:, :D]
    w2 = fc1_w[:, D:]
    b1 = fc1_b.reshape(D, 1)
    pg = pred_w[:, :D]
    pm = pred_w[:, D:]
    pb = pred_b.reshape(1, 1)

    rows_spec = pl.BlockSpec((D, _BLK), lambda i: (0, i))
    w_spec = pl.BlockSpec((D, D), lambda i: (0, 0))
    b_spec = pl.BlockSpec((D, 1), lambda i: (0, 0))
    v_spec = pl.BlockSpec((1, D), lambda i: (0, 0))
    s_spec = pl.BlockSpec((1, 1), lambda i: (0, 0))
    out2d = pl.pallas_call(
        _tc_body,
        grid=(B // _BLK,),
        in_specs=[rows_spec, rows_spec, rows_spec, rows_spec,
                  w_spec, w_spec, b_spec, v_spec, v_spec, s_spec],
        out_specs=pl.BlockSpec((1, _BLK), lambda i: (0, i)),
        out_shape=jax.ShapeDtypeStruct((1, B), jnp.float32),
    )(gu, gi, mu, mi, w1, w2, b1, pg, pm, pb)
    return out2d.reshape(B)


# speedup vs baseline: 3.8660x; 1.2337x over previous
import functools

import jax
import jax.numpy as jnp
from jax import lax
from jax.experimental import pallas as pl
from jax.experimental.pallas import tpu as pltpu
from jax.experimental.pallas import tpu_sc as plsc

B = 16384
D = 32
V = 1000000
NC, NS = 2, 16
NW = NC * NS
BPW = B // NW
LANE = 128
LAST = (V // LANE) * LANE
TAIL = V - LAST
DEPTH = 4
HALF = BPW // 2


def _sc_gather_body(ut_g, it_g, ut_m, it_m, aug, aig, aum, aim,
                    uidx_hbm, iidx_hbm,
                    gu_out, gi_out, mu_out, mi_out,
                    uidx_v, iidx_v,
                    bug0, bug1, bug2, bug3, big0, big1, big2, big3,
                    bum0, bum1, bum2, bum3, bim0, bim1, bim2, bim3,
                    ou, oi, om, on, sem0, sem1, sem2, sem3):
    wid = lax.axis_index("s") * NC + lax.axis_index("c")
    base = wid * BPW
    pltpu.sync_copy(uidx_hbm.at[pl.ds(base, BPW)], uidx_v.at[pl.ds(0, BPW)])
    pltpu.sync_copy(iidx_hbm.at[pl.ds(base, BPW)], iidx_v.at[pl.ds(0, BPW)])

    def fetch(tab, aux, off, blk, sem, start):
        @pl.when(off < LAST)
        def _full():
            o = pl.multiple_of(off, LANE)
            cp = pltpu.make_async_copy(tab.at[:, pl.ds(o, LANE)], blk, sem)
            cp.start() if start else cp.wait()

        @pl.when(off >= LAST)
        def _tail():
            cp = pltpu.make_async_copy(aux, blk, sem)
            cp.start() if start else cp.wait()

    def quad(u, i, blks, sem, start):
        uo = u & ~(LANE - 1)
        io = i & ~(LANE - 1)
        fetch(ut_g, aug, uo, blks[0], sem, start)
        fetch(it_g, aig, io, blks[1], sem, start)
        fetch(ut_m, aum, uo, blks[2], sem, start)
        fetch(it_m, aim, io, blks[3], sem, start)

    def extract(u, i, j, blks):
        rows = lax.iota(jnp.int32, 16)
        ul = jnp.full((16,), u & (LANE - 1), jnp.int32)
        il = jnp.full((16,), i & (LANE - 1), jnp.int32)
        col = jnp.full((16,), j & (HALF - 1), jnp.int32)
        for blk, lane, out in ((blks[0], ul, ou), (blks[1], il, oi),
                               (blks[2], ul, om), (blks[3], il, on)):
            for half in (0, 16):
                r = rows + half
                v = plsc.load_gather(blk, [r, lane])
                plsc.store_scatter(out, [r, col], v)

    phs = ((bug0, big0, bum0, bim0), (bug1, big1, bum1, bim1),
           (bug2, big2, bum2, bim2), (bug3, big3, bum3, bim3))
    sems = (sem0, sem1, sem2, sem3)

    def flush(half_base):
        for buf, out in ((ou, gu_out), (oi, gi_out),
                         (om, mu_out), (on, mi_out)):
            pltpu.sync_copy(buf, out.at[:, pl.ds(half_base, HALF)])

    uv0 = uidx_v[pl.ds(0, 16)]
    iv0 = iidx_v[pl.ds(0, 16)]
    for p in range(DEPTH):
        quad(uv0[p], iv0[p], phs[p], sems[p], True)

    def step(j2, _):
        j0 = j2 * DEPTH

        @pl.when(j0 == HALF)
        def _mid():
            flush(base)

        uv = uidx_v[pl.ds(j0, 16)]
        iv = iidx_v[pl.ds(j0, 16)]
        for p in range(DEPTH):
            quad(uv[p], iv[p], phs[p], sems[p], False)
            extract(uv[p], iv[p], j0 + p, phs[p])

            @pl.when(j0 + p + DEPTH < BPW)
            def _issue():
                quad(uv[p + DEPTH], iv[p + DEPTH], phs[p], sems[p], True)
        return 0

    lax.fori_loop(0, BPW // DEPTH, step, 0)
    flush(base + HALF)


_FM = jax.ShapeDtypeStruct((D, B), jnp.float32)
_BLKT = pltpu.VMEM((D, LANE), jnp.float32)

_sc_gather = functools.partial(
    pl.kernel,
    out_type=(_FM, _FM, _FM, _FM),
    mesh=plsc.VectorSubcoreMesh(
        core_axis_name="c", subcore_axis_name="s",
        num_cores=NC, num_subcores=NS),
    scratch_types=(
        pltpu.VMEM((BPW + 16,), jnp.int32),
        pltpu.VMEM((BPW + 16,), jnp.int32),
        _BLKT, _BLKT, _BLKT, _BLKT, _BLKT, _BLKT, _BLKT, _BLKT,
        _BLKT, _BLKT, _BLKT, _BLKT, _BLKT, _BLKT, _BLKT, _BLKT,
        pltpu.VMEM((D, HALF), jnp.float32),
        pltpu.VMEM((D, HALF), jnp.float32),
        pltpu.VMEM((D, HALF), jnp.float32),
        pltpu.VMEM((D, HALF), jnp.float32),
        pltpu.SemaphoreType.DMA,
        pltpu.SemaphoreType.DMA,
        pltpu.SemaphoreType.DMA,
        pltpu.SemaphoreType.DMA,
    ),
    compiler_params=pltpu.CompilerParams(needs_layout_passes=False),
)(_sc_gather_body)


def _tc_body(gu_ref, gi_ref, mu_ref, mi_ref, w1_ref, w2_ref, b1_ref,
             pg_ref, pm_ref, pb_ref, out_ref):
    h = jnp.dot(w1_ref[...], mu_ref[...], preferred_element_type=jnp.float32)
    h = h + jnp.dot(w2_ref[...], mi_ref[...],
                    preferred_element_type=jnp.float32)
    h = jnp.maximum(h + b1_ref[...], 0.0)
    gmf = gu_ref[...] * gi_ref[...]
    out_ref[...] = (jnp.dot(pg_ref[...], gmf,
                            preferred_element_type=jnp.float32)
                    + jnp.dot(pm_ref[...], h,
                              preferred_element_type=jnp.float32)
                    + pb_ref[...])


_BLK = 2048


def kernel(user_indices, item_indices, gmf_user_emb, gmf_item_emb,
           mlp_user_emb, mlp_item_emb, fc1_w, fc1_b, pred_w, pred_b):
    uidx = user_indices.astype(jnp.int32)
    iidx = item_indices.astype(jnp.int32)
    def _aux(tab):
        return jnp.pad(tab.T[:, LAST:], ((0, 0), (0, LANE - TAIL)))

    gu, gi, mu, mi = _sc_gather(
        gmf_user_emb.T, gmf_item_emb.T, mlp_user_emb.T, mlp_item_emb.T,
        _aux(gmf_user_emb), _aux(gmf_item_emb),
        _aux(mlp_user_emb), _aux(mlp_item_emb), uidx, iidx)

    w1 = fc1_w[:, :D]
    w2 = fc1_w[:, D:]
    b1 = fc1_b.reshape(D, 1)
    pg = pred_w[:, :D]
    pm = pred_w[:, D:]
    pb = pred_b.reshape(1, 1)

    rows_spec = pl.BlockSpec((D, _BLK), lambda i: (0, i))
    w_spec = pl.BlockSpec((D, D), lambda i: (0, 0))
    b_spec = pl.BlockSpec((D, 1), lambda i: (0, 0))
    v_spec = pl.BlockSpec((1, D), lambda i: (0, 0))
    s_spec = pl.BlockSpec((1, 1), lambda i: (0, 0))
    out2d = pl.pallas_call(
        _tc_body,
        grid=(B // _BLK,),
        in_specs=[rows_spec, rows_spec, rows_spec, rows_spec,
                  w_spec, w_spec, b_spec, v_spec, v_spec, s_spec],
        out_specs=pl.BlockSpec((1, _BLK), lambda i: (0, i)),
        out_shape=jax.ShapeDtypeStruct((1, B), jnp.float32),
    )(gu, gi, mu, mi, w1, w2, b1, pg, pm, pb)
    return out2d.reshape(B)


# speedup vs baseline: 6.4837x; 1.6771x over previous
import functools

import jax
import jax.numpy as jnp
from jax import lax
from jax.experimental import pallas as pl
from jax.experimental.pallas import tpu as pltpu
from jax.experimental.pallas import tpu_sc as plsc

B = 16384
D = 32
V = 1000000
NC, NS = 2, 16
NW = NC * NS
BPW = B // NW
LANE = 128
LAST = (V // LANE) * LANE
TAIL = V - LAST
DEPTH = 4
CHK = 128
NCHK = BPW // CHK
ROWW = 128


def _sc_gather_body(ut_g, ut_m, it_g, it_m, aug, aum, aig, aim,
                    us_hbm, up_hbm, is_hbm, ip_hbm,
                    urows_hbm, irows_hbm,
                    sv, pv, rs, rows,
                    b00, b01, b10, b11, b20, b21, b30, b31,
                    sem0, sem1, sem2, sem3, ssem):
    wid = lax.axis_index("s") * NC + lax.axis_index("c")
    base = wid * BPW
    phs = ((b00, b01), (b10, b11), (b20, b21), (b30, b31))
    sems = (sem0, sem1, sem2, sem3)
    i16 = lax.iota(jnp.int32, 16)

    def fetch(tab, aux, off, blk, sem, start):
        @pl.when(off < LAST)
        def _full():
            o = pl.multiple_of(off, LANE)
            cp = pltpu.make_async_copy(tab.at[:, pl.ds(o, LANE)], blk, sem)
            cp.start() if start else cp.wait()

        @pl.when(off >= LAST)
        def _tail():
            cp = pltpu.make_async_copy(aux, blk, sem)
            cp.start() if start else cp.wait()

    def one_pass(tab_a, tab_b, aux_a, aux_b, s_hbm, p_hbm, out_hbm):
        pltpu.sync_copy(s_hbm.at[pl.ds(base, BPW)], sv.at[pl.ds(8, BPW)])
        pltpu.sync_copy(p_hbm.at[pl.ds(wid * NCHK, NCHK)], pv)

        cur0 = sv[pl.ds(8, 16)]
        prv0 = sv[pl.ds(7, 16)]
        m0 = ((cur0 >> 7) != (prv0 >> 7)) | (i16 == 0)
        plsc.store_compressed(rs.at[pl.ds(0, 16)], i16, mask=m0)
        n0 = plsc.all_reduce_population_count(m0)[0]

        def chunk(o, off):
            o16 = o * 16
            cur = sv[pl.ds(o16 + 8, 16)]
            prv = sv[pl.ds(o16 + 7, 16)]
            m = (cur >> 7) != (prv >> 7)
            plsc.store_compressed(rs.at[pl.ds(off, 16)], i16 + o16, mask=m)
            return off + plsc.all_reduce_population_count(m)[0]

        n_run = lax.fori_loop(1, BPW // 16, chunk, n0)
        plsc.store_compressed(rs.at[pl.ds(n_run, 16)],
                              jnp.full((16,), BPW, jnp.int32), mask=i16 == 0)

        def fpair(f, blks, sem, start):
            st = rs[pl.ds(f, 16)][0]
            u = sv[pl.ds(st + 8, 16)][0]
            off = u & ~(LANE - 1)
            fetch(tab_a, aux_a, off, blks[0], sem, start)
            fetch(tab_b, aux_b, off, blks[1], sem, start)

        for p in range(DEPTH):
            @pl.when(p < n_run)
            def _pro():
                fpair(p, phs[p], sems[p], True)

        def outer(t, _):
            f0 = t * DEPTH
            for p in range(DEPTH):
                f = f0 + p

                @pl.when(f < n_run)
                def _do():
                    fpair(f, phs[p], sems[p], False)
                    rsv = rs[pl.ds(f, 16)]

                    def ex(j, _c):
                        u = sv[pl.ds(j + 8, 16)][0]
                        l = jnp.full((16,), u & (LANE - 1), jnp.int32)
                        rows[j, pl.ds(0, 16)] = plsc.load_gather(
                            phs[p][0], [i16, l])
                        rows[j, pl.ds(16, 16)] = plsc.load_gather(
                            phs[p][0], [i16 + 16, l])
                        rows[j, pl.ds(32, 16)] = plsc.load_gather(
                            phs[p][1], [i16, l])
                        rows[j, pl.ds(48, 16)] = plsc.load_gather(
                            phs[p][1], [i16 + 16, l])
                        return 0

                    lax.fori_loop(rsv[0], rsv[1], ex, 0)

                    @pl.when(f + DEPTH < n_run)
                    def _nxt():
                        fpair(f + DEPTH, phs[p], sems[p], True)
            return 0

        lax.fori_loop(0, (n_run + DEPTH - 1) >> 2, outer, 0)

        cps = []
        for c in range(NCHK):
            cps.append(pltpu.async_copy(
                rows.at[pl.ds(c * CHK, CHK)], out_hbm.at[pv.at[c]], ssem))
        for cp in cps:
            cp.wait()

    one_pass(ut_g, ut_m, aug, aum, us_hbm, up_hbm, urows_hbm)
    one_pass(it_g, it_m, aig, aim, is_hbm, ip_hbm, irows_hbm)


_ROWS = jax.ShapeDtypeStruct((B, ROWW), jnp.float32)
_BLKT = pltpu.VMEM((D, LANE), jnp.float32)

_sc_gather = functools.partial(
    pl.kernel,
    out_type=(_ROWS, _ROWS),
    mesh=plsc.VectorSubcoreMesh(
        core_axis_name="c", subcore_axis_name="s",
        num_cores=NC, num_subcores=NS),
    scratch_types=(
        pltpu.VMEM((BPW + 32,), jnp.int32),
        pltpu.VMEM((NCHK, CHK), jnp.int32),
        pltpu.VMEM((BPW + 32,), jnp.int32),
        pltpu.VMEM((BPW, ROWW), jnp.float32),
        _BLKT, _BLKT, _BLKT, _BLKT, _BLKT, _BLKT, _BLKT, _BLKT,
        pltpu.SemaphoreType.DMA,
        pltpu.SemaphoreType.DMA,
        pltpu.SemaphoreType.DMA,
        pltpu.SemaphoreType.DMA,
        pltpu.SemaphoreType.DMA,
    ),
    compiler_params=pltpu.CompilerParams(needs_layout_passes=False),
)(_sc_gather_body)


def _tc_body(ur_ref, ir_ref, w1_ref, w2_ref, b1_ref,
             pg_ref, pm_ref, pb_ref, out_ref):
    ur = ur_ref[...]
    ir = ir_ref[...]
    gu, mu = ur[:, :D], ur[:, D:2 * D]
    gi, mi = ir[:, :D], ir[:, D:2 * D]
    h = jnp.dot(mu, w1_ref[...], preferred_element_type=jnp.float32)
    h = h + jnp.dot(mi, w2_ref[...], preferred_element_type=jnp.float32)
    h = jnp.maximum(h + b1_ref[...], 0.0)
    gmf = gu * gi
    out_ref[...] = (jnp.dot(gmf, pg_ref[...],
                            preferred_element_type=jnp.float32)
                    + jnp.dot(h, pm_ref[...],
                              preferred_element_type=jnp.float32)
                    + pb_ref[...])


_BLK = 2048


def kernel(user_indices, item_indices, gmf_user_emb, gmf_item_emb,
           mlp_user_emb, mlp_item_emb, fc1_w, fc1_b, pred_w, pred_b):
    uidx = user_indices.astype(jnp.int32)
    iidx = item_indices.astype(jnp.int32)
    up = jnp.argsort(uidx)
    ip = jnp.argsort(iidx)
    us = jnp.take(uidx, up)
    isrt = jnp.take(iidx, ip)

    def _aux(tab):
        return jnp.pad(tab.T[:, LAST:], ((0, 0), (0, LANE - TAIL)))

    urows, irows = _sc_gather(
        gmf_user_emb.T, mlp_user_emb.T, gmf_item_emb.T, mlp_item_emb.T,
        _aux(gmf_user_emb), _aux(mlp_user_emb),
        _aux(gmf_item_emb), _aux(mlp_item_emb),
        us, up.reshape(B // CHK, CHK), isrt, ip.reshape(B // CHK, CHK))

    w1 = fc1_w[:, :D].T
    w2 = fc1_w[:, D:].T
    b1 = fc1_b.reshape(1, D)
    pg = pred_w[:, :D].T
    pm = pred_w[:, D:].T
    pb = pred_b.reshape(1, 1)

    rows_spec = pl.BlockSpec((_BLK, ROWW), lambda i: (i, 0))
    w_spec = pl.BlockSpec((D, D), lambda i: (0, 0))
    r_spec = pl.BlockSpec((1, D), lambda i: (0, 0))
    c_spec = pl.BlockSpec((D, 1), lambda i: (0, 0))
    s_spec = pl.BlockSpec((1, 1), lambda i: (0, 0))
    out2d = pl.pallas_call(
        _tc_body,
        grid=(B // _BLK,),
        in_specs=[rows_spec, rows_spec,
                  w_spec, w_spec, r_spec, c_spec, c_spec, s_spec],
        out_specs=pl.BlockSpec((_BLK, 1), lambda i: (i, 0)),
        out_shape=jax.ShapeDtypeStruct((B, 1), jnp.float32),
    )(urows, irows, w1, w2, b1, pg, pm, pb)
    return out2d.reshape(B)


# speedup vs baseline: 7.1184x; 1.0979x over previous
import functools

import jax
import jax.numpy as jnp
from jax import lax
from jax.experimental import pallas as pl
from jax.experimental.pallas import tpu as pltpu
from jax.experimental.pallas import tpu_sc as plsc

B = 16384
D = 32
V = 1000000
NC, NS = 2, 16
NW = NC * NS
BPW = B // NW
LANE = 128
LAST = (V // LANE) * LANE
TAIL = V - LAST
DEPTH = 6
CHK = 128
NCHK = BPW // CHK
ROWW = 128


def _sc_gather_body(ut_g, ut_m, it_g, it_m, aug, aum, aig, aim,
                    us_hbm, up_hbm, is_hbm, ip_hbm,
                    urows_hbm, irows_hbm,
                    sv, pv, rs, rows,
                    b00, b01, b10, b11, b20, b21, b30, b31,
                    b40, b41, b50, b51,
                    sem0, sem1, sem2, sem3, sem4, sem5, ssem):
    wid = lax.axis_index("s") * NC + lax.axis_index("c")
    base = wid * BPW
    phs = ((b00, b01), (b10, b11), (b20, b21), (b30, b31),
           (b40, b41), (b50, b51))
    sems = (sem0, sem1, sem2, sem3, sem4, sem5)
    i16 = lax.iota(jnp.int32, 16)

    def fetch(tab, aux, off, blk, sem, start):
        @pl.when(off < LAST)
        def _full():
            o = pl.multiple_of(off, LANE)
            cp = pltpu.make_async_copy(tab.at[:, pl.ds(o, LANE)], blk, sem)
            cp.start() if start else cp.wait()

        @pl.when(off >= LAST)
        def _tail():
            cp = pltpu.make_async_copy(aux, blk, sem)
            cp.start() if start else cp.wait()

    def one_pass(tab_a, tab_b, aux_a, aux_b, s_hbm, p_hbm, out_hbm):
        pltpu.sync_copy(s_hbm.at[pl.ds(base, BPW)], sv.at[pl.ds(8, BPW)])
        pltpu.sync_copy(p_hbm.at[pl.ds(wid * NCHK, NCHK)], pv)

        cur0 = sv[pl.ds(8, 16)]
        prv0 = sv[pl.ds(7, 16)]
        m0 = ((cur0 >> 7) != (prv0 >> 7)) | (i16 == 0)
        plsc.store_compressed(rs.at[pl.ds(0, 16)], i16, mask=m0)
        n0 = plsc.all_reduce_population_count(m0)[0]

        def chunk(o, off):
            o16 = o * 16
            cur = sv[pl.ds(o16 + 8, 16)]
            prv = sv[pl.ds(o16 + 7, 16)]
            m = (cur >> 7) != (prv >> 7)
            plsc.store_compressed(rs.at[pl.ds(off, 16)], i16 + o16, mask=m)
            return off + plsc.all_reduce_population_count(m)[0]

        n_run = lax.fori_loop(1, BPW // 16, chunk, n0)
        plsc.store_compressed(rs.at[pl.ds(n_run, 16)],
                              jnp.full((16,), BPW, jnp.int32), mask=i16 == 0)

        def fpair(f, blks, sem, start):
            st = rs[pl.ds(f, 16)][0]
            u = sv[pl.ds(st + 8, 16)][0]
            off = u & ~(LANE - 1)
            fetch(tab_a, aux_a, off, blks[0], sem, start)
            fetch(tab_b, aux_b, off, blks[1], sem, start)

        for p in range(DEPTH):
            @pl.when(p < n_run)
            def _pro():
                fpair(p, phs[p], sems[p], True)

        def outer(t, _):
            f0 = t * DEPTH
            for p in range(DEPTH):
                f = f0 + p

                @pl.when(f < n_run)
                def _do():
                    fpair(f, phs[p], sems[p], False)
                    rsv = rs[pl.ds(f, 16)]

                    def ex(j, _c):
                        u = sv[pl.ds(j + 8, 16)][0]
                        l = jnp.full((16,), u & (LANE - 1), jnp.int32)
                        rows[j, pl.ds(0, 16)] = plsc.load_gather(
                            phs[p][0], [i16, l])
                        rows[j, pl.ds(16, 16)] = plsc.load_gather(
                            phs[p][0], [i16 + 16, l])
                        rows[j, pl.ds(32, 16)] = plsc.load_gather(
                            phs[p][1], [i16, l])
                        rows[j, pl.ds(48, 16)] = plsc.load_gather(
                            phs[p][1], [i16 + 16, l])
                        return 0

                    lax.fori_loop(rsv[0], rsv[1], ex, 0)

                    @pl.when(f + DEPTH < n_run)
                    def _nxt():
                        fpair(f + DEPTH, phs[p], sems[p], True)
            return 0

        lax.fori_loop(0, (n_run + DEPTH - 1) // DEPTH, outer, 0)

        cps = []
        for c in range(NCHK):
            cps.append(pltpu.async_copy(
                rows.at[pl.ds(c * CHK, CHK)], out_hbm.at[pv.at[c]], ssem))
        for cp in cps:
            cp.wait()

    one_pass(ut_g, ut_m, aug, aum, us_hbm, up_hbm, urows_hbm)
    one_pass(it_g, it_m, aig, aim, is_hbm, ip_hbm, irows_hbm)


_ROWS = jax.ShapeDtypeStruct((B, ROWW), jnp.float32)
_BLKT = pltpu.VMEM((D, LANE), jnp.float32)

_sc_gather = functools.partial(
    pl.kernel,
    out_type=(_ROWS, _ROWS),
    mesh=plsc.VectorSubcoreMesh(
        core_axis_name="c", subcore_axis_name="s",
        num_cores=NC, num_subcores=NS),
    scratch_types=(
        pltpu.VMEM((BPW + 32,), jnp.int32),
        pltpu.VMEM((NCHK, CHK), jnp.int32),
        pltpu.VMEM((BPW + 32,), jnp.int32),
        pltpu.VMEM((BPW, ROWW), jnp.float32),
        _BLKT, _BLKT, _BLKT, _BLKT, _BLKT, _BLKT, _BLKT, _BLKT,
        _BLKT, _BLKT, _BLKT, _BLKT,
        pltpu.SemaphoreType.DMA,
        pltpu.SemaphoreType.DMA,
        pltpu.SemaphoreType.DMA,
        pltpu.SemaphoreType.DMA,
        pltpu.SemaphoreType.DMA,
        pltpu.SemaphoreType.DMA,
        pltpu.SemaphoreType.DMA,
    ),
    compiler_params=pltpu.CompilerParams(needs_layout_passes=False),
)(_sc_gather_body)


def _tc_body(ur_ref, ir_ref, w1_ref, w2_ref, b1_ref,
             pg_ref, pm_ref, pb_ref, out_ref):
    ur = ur_ref[...]
    ir = ir_ref[...]
    gu, mu = ur[:, :D], ur[:, D:2 * D]
    gi, mi = ir[:, :D], ir[:, D:2 * D]
    h = jnp.dot(mu, w1_ref[...], preferred_element_type=jnp.float32)
    h = h + jnp.dot(mi, w2_ref[...], preferred_element_type=jnp.float32)
    h = jnp.maximum(h + b1_ref[...], 0.0)
    gmf = gu * gi
    out_ref[...] = (jnp.dot(gmf, pg_ref[...],
                            preferred_element_type=jnp.float32)
                    + jnp.dot(h, pm_ref[...],
                              preferred_element_type=jnp.float32)
                    + pb_ref[...])


_BLK = 2048


def kernel(user_indices, item_indices, gmf_user_emb, gmf_item_emb,
           mlp_user_emb, mlp_item_emb, fc1_w, fc1_b, pred_w, pred_b):
    uidx = user_indices.astype(jnp.int32)
    iidx = item_indices.astype(jnp.int32)
    up = jnp.argsort(uidx)
    ip = jnp.argsort(iidx)
    us = jnp.take(uidx, up)
    isrt = jnp.take(iidx, ip)

    def _aux(tab):
        return jnp.pad(tab.T[:, LAST:], ((0, 0), (0, LANE - TAIL)))

    urows, irows = _sc_gather(
        gmf_user_emb.T, mlp_user_emb.T, gmf_item_emb.T, mlp_item_emb.T,
        _aux(gmf_user_emb), _aux(mlp_user_emb),
        _aux(gmf_item_emb), _aux(mlp_item_emb),
        us, up.reshape(B // CHK, CHK), isrt, ip.reshape(B // CHK, CHK))

    w1 = fc1_w[:, :D].T
    w2 = fc1_w[:, D:].T
    b1 = fc1_b.reshape(1, D)
    pg = pred_w[:, :D].T
    pm = pred_w[:, D:].T
    pb = pred_b.reshape(1, 1)

    rows_spec = pl.BlockSpec((_BLK, ROWW), lambda i: (i, 0))
    w_spec = pl.BlockSpec((D, D), lambda i: (0, 0))
    r_spec = pl.BlockSpec((1, D), lambda i: (0, 0))
    c_spec = pl.BlockSpec((D, 1), lambda i: (0, 0))
    s_spec = pl.BlockSpec((1, 1), lambda i: (0, 0))
    out2d = pl.pallas_call(
        _tc_body,
        grid=(B // _BLK,),
        in_specs=[rows_spec, rows_spec,
                  w_spec, w_spec, r_spec, c_spec, c_spec, s_spec],
        out_specs=pl.BlockSpec((_BLK, 1), lambda i: (i, 0)),
        out_shape=jax.ShapeDtypeStruct((B, 1), jnp.float32),
    )(urows, irows, w1, w2, b1, pg, pm, pb)
    return out2d.reshape(B)
